# baseline (speedup 1.0000x reference)
import jax
import jax.numpy as jnp
from jax import lax
from jax.experimental import pallas as pl
from jax.experimental.pallas import tpu as pltpu
from jax.experimental.pallas import tpu_sc as plsc

NUM_NODES = 10000
NUM_EDGES = 320000
D = 128

NC = 2
NS = 16
L = 16
RING = 3
NW = NC * NS
E_PER_W = NUM_EDGES // NW
E_PER_C = NUM_EDGES // NC
CHUNK = 96
G = CHUNK // L
NFULL = E_PER_W // CHUNK
TAIL = E_PER_W - NFULL * CHUNK
NBLK = NUM_NODES // CHUNK
ABLK_TAIL = NUM_NODES - NBLK * CHUNK
BLK_ITERS = (NBLK + NS - 1) // NS
DUMMY = NUM_NODES
TIERS = (8, 16, 32, 64, CHUNK)


def _scale(buf, cut_v, nrows):
    def scale_group(g, inner):
        cvec = cut_v[pl.ds(g * L, L)]
        for r in range(L):
            cs = cvec[r]
            k = g * L + r
            for j in range(D // L):
                buf[k, j * L:(j + 1) * L] = buf[k, j * L:(j + 1) * L] * cs
        return inner

    lax.fori_loop(0, nrows // L, scale_group, 0)


def _zero_rows(buf, nrows):
    zero16 = jnp.zeros((L,), jnp.float32)

    def zero_row(i, carry):
        for j in range(D // L):
            buf[i, j * L:(j + 1) * L] = zero16
        return carry

    lax.fori_loop(0, nrows, zero_row, 0)


def _seg_reduce(buf, cut_v, recv_v, cbuf):
    iota = lax.iota(jnp.int32, L)
    dummy16 = jnp.full((L,), DUMMY, jnp.int32)
    zero16 = jnp.zeros((L,), jnp.float32)

    def group_body(g, carry):
        ss, prev, idxv, accs = carry
        gbase = g * L
        rvec = recv_v[pl.ds(gbase, L)]
        cvec = cut_v[pl.ds(gbase, L)]
        for r in range(L):
            rv = rvec[r]
            cs = cvec[r]
            neq = rv != prev
            ss = ss + jnp.where(neq, jnp.int32(1), jnp.int32(0))
            ks = jnp.where(neq, jnp.float32(0.0), jnp.float32(1.0))
            idxv = jnp.where(iota == ss, rv, idxv)
            ri = gbase + r
            new_accs = []
            for j in range(D // L):
                a = buf[ri, j * L:(j + 1) * L] * cs + accs[j] * ks
                cbuf[ss, j * L:(j + 1) * L] = a
                new_accs.append(a)
            accs = tuple(new_accs)
            prev = rv
        return ss, prev, idxv, accs

    ss, _, idxv, _ = lax.fori_loop(0, G, group_body,
                                   (jnp.int32(-1), jnp.int32(-1), dummy16,
                                    (zero16,) * (D // L)))
    return ss + 1, idxv


def _sc_body(sh_hbm, cut_hbm, recv_hbm, out_hbm, rng_hbm,
             buf0, cut0, recv0, buf1, cut1, recv1, buf2, cut2, recv2,
             cbuf, rbuf,
             acc, fsem0, fsem1, fsem2):
    cid = lax.axis_index("c")
    sid = lax.axis_index("s")
    wid = cid * NS + sid
    ebase0 = wid * E_PER_W

    bufs = (buf0, buf1, buf2)
    cuts = (cut0, cut1, cut2)
    recvs = (recv0, recv1, recv2)
    fsems = (fsem0, fsem1, fsem2)

    pltpu.sync_copy(recv_hbm.at[pl.ds(cid * E_PER_C, L)],
                    recv0.at[pl.ds(0, L)])
    lo = recv0[pl.ds(0, L)][0]
    pltpu.sync_copy(recv_hbm.at[pl.ds((cid + 1) * E_PER_C - L, L)],
                    recv0.at[pl.ds(0, L)])
    hi = recv0[pl.ds(0, L)][L - 1]

    def blk_touched(b):
        return jnp.logical_and(b * CHUNK <= hi, b * CHUNK + CHUNK > lo)

    tail_touched = NBLK * CHUNK <= hi

    _zero_rows(buf0, CHUNK)
    for t in range(BLK_ITERS):
        b = t * NS + sid

        @pl.when(jnp.logical_and(b < NBLK, blk_touched(b)))
        def _():
            pltpu.sync_copy(buf0, acc.at[pl.ds(b * CHUNK, CHUNK)])

    @pl.when(jnp.logical_and(sid == 0, tail_touched))
    def _():
        pltpu.sync_copy(buf0.at[pl.ds(0, ABLK_TAIL)],
                        acc.at[pl.ds(NBLK * CHUNK, ABLK_TAIL)])

    @pl.when(sid == 0)
    def _():
        idx = lax.iota(jnp.int32, L)
        lo_v = jnp.broadcast_to(lo, (L,)).astype(jnp.int32)
        hi_v = jnp.broadcast_to(hi, (L,)).astype(jnp.int32)
        rbuf[pl.ds(0, L)] = jnp.where(idx == 0, lo_v, hi_v)
        pltpu.sync_copy(rbuf, rng_hbm.at[pl.ds(cid * L, L)])

    plsc.subcore_barrier()

    def fill(k, j):
        e = ebase0 + j * CHUNK
        pltpu.async_copy(sh_hbm.at[pl.ds(e, CHUNK)], bufs[k], fsems[k])
        pltpu.async_copy(cut_hbm.at[pl.ds(e, CHUNK)], cuts[k], fsems[k])
        pltpu.async_copy(recv_hbm.at[pl.ds(e, CHUNK)], recvs[k], fsems[k])

    def wait_fill(k, j):
        e = ebase0 + j * CHUNK
        pltpu.make_async_copy(sh_hbm.at[pl.ds(e, CHUNK)], bufs[k],
                              fsems[k]).wait()
        pltpu.make_async_copy(cut_hbm.at[pl.ds(e, CHUNK)], cuts[k],
                              fsems[k]).wait()
        pltpu.make_async_copy(recv_hbm.at[pl.ds(e, CHUNK)], recvs[k],
                              fsems[k]).wait()

    def do_chunk(j, k):
        wait_fill(k, j)
        nseg, idxv = _seg_reduce(bufs[k], cuts[k], recvs[k], cbuf)

        @pl.when(nseg <= L)
        def _():
            pltpu.sync_copy(cbuf.at[pl.ds(0, L)], acc.at[idxv], add=True)

        @pl.when(nseg > L)
        def _():
            _scale(bufs[k], cuts[k], CHUNK)
            pltpu.sync_copy(bufs[k], acc.at[recvs[k]], add=True)

        @pl.when(j + RING < NFULL)
        def _():
            fill(k, j + RING)

    for j0 in range(RING):
        fill(j0, j0)

    def round_body(t, carry):
        for k in range(RING):
            do_chunk(RING * t + k, k)
        return carry

    ROUNDS = NFULL // RING
    lax.fori_loop(0, ROUNDS, round_body, 0)
    for j in range(RING * ROUNDS, NFULL):
        do_chunk(j, j % RING)

    etail = ebase0 + NFULL * CHUNK
    pltpu.sync_copy(sh_hbm.at[pl.ds(etail, TAIL)], buf0.at[pl.ds(0, TAIL)])
    pltpu.sync_copy(cut_hbm.at[pl.ds(etail, TAIL)], cut0.at[pl.ds(0, TAIL)])
    pltpu.sync_copy(recv_hbm.at[pl.ds(etail, TAIL)], recv0.at[pl.ds(0, TAIL)])
    _scale(buf0, cut0, TAIL)
    pltpu.sync_copy(buf0.at[pl.ds(0, TAIL)],
                    acc.at[recv0.at[pl.ds(0, TAIL)]], add=True)

    plsc.subcore_barrier()

    obase = cid * NUM_NODES
    for t in range(BLK_ITERS):
        b = t * NS + sid

        @pl.when(jnp.logical_and(b < NBLK, blk_touched(b)))
        def _():
            pltpu.sync_copy(acc.at[pl.ds(b * CHUNK, CHUNK)], buf0)
            pltpu.sync_copy(buf0, out_hbm.at[pl.ds(obase + b * CHUNK, CHUNK)])

    @pl.when(jnp.logical_and(sid == 0, tail_touched))
    def _():
        pltpu.sync_copy(acc.at[pl.ds(NBLK * CHUNK, ABLK_TAIL)],
                        buf0.at[pl.ds(0, ABLK_TAIL)])
        pltpu.sync_copy(buf0.at[pl.ds(0, ABLK_TAIL)],
                        out_hbm.at[pl.ds(obase + NBLK * CHUNK, ABLK_TAIL)])


_sc_scatter = pl.kernel(
    _sc_body,
    out_type=(
        jax.ShapeDtypeStruct((NC * NUM_NODES, D), jnp.float32),
        jax.ShapeDtypeStruct((NC * L,), jnp.int32),
    ),
    mesh=plsc.VectorSubcoreMesh(core_axis_name="c", subcore_axis_name="s"),
    scratch_types=[
        pltpu.VMEM((CHUNK, D), jnp.float32),
        pltpu.VMEM((CHUNK,), jnp.float32),
        pltpu.VMEM((CHUNK,), jnp.int32),
        pltpu.VMEM((CHUNK, D), jnp.float32),
        pltpu.VMEM((CHUNK,), jnp.float32),
        pltpu.VMEM((CHUNK,), jnp.int32),
        pltpu.VMEM((CHUNK, D), jnp.float32),
        pltpu.VMEM((CHUNK,), jnp.float32),
        pltpu.VMEM((CHUNK,), jnp.int32),
        pltpu.VMEM((CHUNK, D), jnp.float32),
        pltpu.VMEM((L,), jnp.int32),
        pltpu.VMEM_SHARED((NUM_NODES + 8, D), jnp.float32),
        pltpu.SemaphoreType.DMA,
        pltpu.SemaphoreType.DMA,
        pltpu.SemaphoreType.DMA,
    ],
)


def _combine_body(inv_ref, rng_ref, p_ref, o_ref):
    i = pl.program_id(0)
    rows = i * _COMBINE_BLK + lax.broadcasted_iota(
        jnp.int32, (_COMBINE_BLK, D), 0)
    lo0, hi0 = rng_ref[0], rng_ref[1]
    lo1, hi1 = rng_ref[L], rng_ref[L + 1]
    m0 = jnp.logical_and(rows >= lo0, rows <= hi0)
    m1 = jnp.logical_and(rows >= lo1, rows <= hi1)
    zero = jnp.zeros_like(o_ref)
    p0 = jnp.where(m0, p_ref[0], zero)
    p1 = jnp.where(m1, p_ref[1], zero)
    o_ref[...] = (p0 + p1) * inv_ref[0]


_COMBINE_BLK = 1000


def _combine(partials, rng, inv_arr):
    return pl.pallas_call(
        _combine_body,
        grid=(NUM_NODES // _COMBINE_BLK,),
        in_specs=[
            pl.BlockSpec(memory_space=pltpu.SMEM),
            pl.BlockSpec(memory_space=pltpu.SMEM),
            pl.BlockSpec((NC, _COMBINE_BLK, D), lambda i: (0, i, 0)),
        ],
        out_specs=pl.BlockSpec((_COMBINE_BLK, D), lambda i: (i, 0)),
        out_shape=jax.ShapeDtypeStruct((NUM_NODES, D), jnp.float32),
    )(inv_arr, rng, partials)


def kernel(sh_vectors, cutoffs, receivers, inv_avg_num_neighbors):
    recv32 = receivers.astype(jnp.int32)
    cut_flat = cutoffs.reshape(NUM_EDGES)
    partials, rng = _sc_scatter(sh_vectors, cut_flat, recv32)
    inv_arr = jnp.reshape(inv_avg_num_neighbors, (1,)).astype(jnp.float32)
    return _combine(partials.reshape(NC, NUM_NODES, D), rng, inv_arr)


# speedup vs baseline: 3.2627x; 3.2627x over previous
import jax
import jax.numpy as jnp
from jax import lax
from jax.experimental import pallas as pl
from jax.experimental.pallas import tpu as pltpu
from jax.experimental.pallas import tpu_sc as plsc

NUM_NODES = 10000
NUM_EDGES = 320000
D = 128

NC = 2
NS = 16
L = 16
RING = 4
NW = NC * NS
E_PER_W = NUM_EDGES // NW
E_PER_C = NUM_EDGES // NC
CHUNK = 96
NFULL = E_PER_W // CHUNK
TAIL = E_PER_W - NFULL * CHUNK
NBLK = NUM_NODES // CHUNK
ABLK_TAIL = NUM_NODES - NBLK * CHUNK
BLK_ITERS = (NBLK + NS - 1) // NS


def _scale(buf, cut_v, nrows):
    def scale_group(g, inner):
        cvec = cut_v[pl.ds(g * L, L)]
        for r in range(L):
            cs = cvec[r]
            k = g * L + r
            for j in range(D // L):
                buf[k, j * L:(j + 1) * L] = buf[k, j * L:(j + 1) * L] * cs
        return inner

    lax.fori_loop(0, nrows // L, scale_group, 0)


def _zero_rows(buf, nrows):
    zero16 = jnp.zeros((L,), jnp.float32)

    def zero_row(i, carry):
        for j in range(D // L):
            buf[i, j * L:(j + 1) * L] = zero16
        return carry

    lax.fori_loop(0, nrows, zero_row, 0)


def _sc_body(sh_hbm, cut_hbm, recv_hbm, out_hbm, rng_hbm,
             buf0, cut0, recv0, buf1, cut1, recv1,
             buf2, cut2, recv2, buf3, cut3, recv3, rbuf,
             acc, fsem0, fsem1, fsem2, fsem3, ssem0, ssem1, ssem2, ssem3):
    cid = lax.axis_index("c")
    sid = lax.axis_index("s")
    wid = cid * NS + sid
    ebase0 = wid * E_PER_W

    bufs = (buf0, buf1, buf2, buf3)
    cuts = (cut0, cut1, cut2, cut3)
    recvs = (recv0, recv1, recv2, recv3)
    fsems = (fsem0, fsem1, fsem2, fsem3)
    ssems = (ssem0, ssem1, ssem2, ssem3)

    pltpu.sync_copy(recv_hbm.at[pl.ds(cid * E_PER_C, L)],
                    recv0.at[pl.ds(0, L)])
    lo = recv0[pl.ds(0, L)][0]
    pltpu.sync_copy(recv_hbm.at[pl.ds((cid + 1) * E_PER_C - L, L)],
                    recv0.at[pl.ds(0, L)])
    hi = recv0[pl.ds(0, L)][L - 1]

    def blk_touched(b):
        return jnp.logical_and(b * CHUNK <= hi, b * CHUNK + CHUNK > lo)

    tail_touched = NBLK * CHUNK <= hi

    _zero_rows(buf0, CHUNK)
    for t in range(BLK_ITERS):
        b = t * NS + sid

        @pl.when(jnp.logical_and(b < NBLK, blk_touched(b)))
        def _():
            pltpu.sync_copy(buf0, acc.at[pl.ds(b * CHUNK, CHUNK)])

    @pl.when(jnp.logical_and(sid == 0, tail_touched))
    def _():
        pltpu.sync_copy(buf0.at[pl.ds(0, ABLK_TAIL)],
                        acc.at[pl.ds(NBLK * CHUNK, ABLK_TAIL)])

    @pl.when(sid == 0)
    def _():
        idx = lax.iota(jnp.int32, L)
        lo_v = jnp.broadcast_to(lo, (L,)).astype(jnp.int32)
        hi_v = jnp.broadcast_to(hi, (L,)).astype(jnp.int32)
        rbuf[pl.ds(0, L)] = jnp.where(idx == 0, lo_v, hi_v)
        pltpu.sync_copy(rbuf, rng_hbm.at[pl.ds(cid * L, L)])

    plsc.subcore_barrier()

    def fill(k, j):
        e = ebase0 + j * CHUNK
        pltpu.async_copy(sh_hbm.at[pl.ds(e, CHUNK)], bufs[k], fsems[k])
        pltpu.async_copy(cut_hbm.at[pl.ds(e, CHUNK)], cuts[k], fsems[k])
        pltpu.async_copy(recv_hbm.at[pl.ds(e, CHUNK)], recvs[k], fsems[k])

    def wait_fill(k, j):
        e = ebase0 + j * CHUNK
        pltpu.make_async_copy(sh_hbm.at[pl.ds(e, CHUNK)], bufs[k],
                              fsems[k]).wait()
        pltpu.make_async_copy(cut_hbm.at[pl.ds(e, CHUNK)], cuts[k],
                              fsems[k]).wait()
        pltpu.make_async_copy(recv_hbm.at[pl.ds(e, CHUNK)], recvs[k],
                              fsems[k]).wait()

    def scat(k):
        pltpu.async_copy(bufs[k], acc.at[recvs[k]], ssems[k], add=True)

    def wait_scat(k):
        pltpu.make_async_copy(bufs[k], acc.at[recvs[k]], ssems[k]).wait()

    for j0 in range(RING - 1):
        fill(j0, j0)

    def round_body(t, carry):
        for k in range(RING):
            j = RING * t + k
            wait_fill(k, j)
            kf = (k + RING - 1) % RING
            if k == 0:
                @pl.when(j >= 1)
                def _():
                    wait_scat(kf)
            else:
                wait_scat(kf)

            @pl.when(j + RING - 1 < NFULL)
            def _():
                fill(kf, j + RING - 1)

            _scale(bufs[k], cuts[k], CHUNK)
            scat(k)
        return carry

    lax.fori_loop(0, NFULL // RING, round_body, 0)
    wait_scat((NFULL - 1) % RING)

    etail = ebase0 + NFULL * CHUNK
    pltpu.sync_copy(sh_hbm.at[pl.ds(etail, TAIL)], buf0.at[pl.ds(0, TAIL)])
    pltpu.sync_copy(cut_hbm.at[pl.ds(etail, TAIL)], cut0.at[pl.ds(0, TAIL)])
    pltpu.sync_copy(recv_hbm.at[pl.ds(etail, TAIL)], recv0.at[pl.ds(0, TAIL)])
    _scale(buf0, cut0, TAIL)
    pltpu.sync_copy(buf0.at[pl.ds(0, TAIL)],
                    acc.at[recv0.at[pl.ds(0, TAIL)]], add=True)

    plsc.subcore_barrier()

    obase = cid * NUM_NODES
    for t in range(BLK_ITERS):
        b = t * NS + sid

        @pl.when(jnp.logical_and(b < NBLK, blk_touched(b)))
        def _():
            pltpu.sync_copy(acc.at[pl.ds(b * CHUNK, CHUNK)], buf0)
            pltpu.sync_copy(buf0, out_hbm.at[pl.ds(obase + b * CHUNK, CHUNK)])

    @pl.when(jnp.logical_and(sid == 0, tail_touched))
    def _():
        pltpu.sync_copy(acc.at[pl.ds(NBLK * CHUNK, ABLK_TAIL)],
                        buf0.at[pl.ds(0, ABLK_TAIL)])
        pltpu.sync_copy(buf0.at[pl.ds(0, ABLK_TAIL)],
                        out_hbm.at[pl.ds(obase + NBLK * CHUNK, ABLK_TAIL)])


_sc_scatter = pl.kernel(
    _sc_body,
    out_type=(
        jax.ShapeDtypeStruct((NC * NUM_NODES, D), jnp.float32),
        jax.ShapeDtypeStruct((NC * L,), jnp.int32),
    ),
    mesh=plsc.VectorSubcoreMesh(core_axis_name="c", subcore_axis_name="s"),
    scratch_types=[
        pltpu.VMEM((CHUNK, D), jnp.float32),
        pltpu.VMEM((CHUNK,), jnp.float32),
        pltpu.VMEM((CHUNK,), jnp.int32),
        pltpu.VMEM((CHUNK, D), jnp.float32),
        pltpu.VMEM((CHUNK,), jnp.float32),
        pltpu.VMEM((CHUNK,), jnp.int32),
        pltpu.VMEM((CHUNK, D), jnp.float32),
        pltpu.VMEM((CHUNK,), jnp.float32),
        pltpu.VMEM((CHUNK,), jnp.int32),
        pltpu.VMEM((CHUNK, D), jnp.float32),
        pltpu.VMEM((CHUNK,), jnp.float32),
        pltpu.VMEM((CHUNK,), jnp.int32),
        pltpu.VMEM((L,), jnp.int32),
        pltpu.VMEM_SHARED((NUM_NODES, D), jnp.float32),
        pltpu.SemaphoreType.DMA,
        pltpu.SemaphoreType.DMA,
        pltpu.SemaphoreType.DMA,
        pltpu.SemaphoreType.DMA,
        pltpu.SemaphoreType.DMA,
        pltpu.SemaphoreType.DMA,
        pltpu.SemaphoreType.DMA,
        pltpu.SemaphoreType.DMA,
    ],
)


def _combine_body(inv_ref, rng_ref, p_ref, o_ref):
    i = pl.program_id(0)
    rows = i * _COMBINE_BLK + lax.broadcasted_iota(
        jnp.int32, (_COMBINE_BLK, D), 0)
    lo0, hi0 = rng_ref[0], rng_ref[1]
    lo1, hi1 = rng_ref[L], rng_ref[L + 1]
    m0 = jnp.logical_and(rows >= lo0, rows <= hi0)
    m1 = jnp.logical_and(rows >= lo1, rows <= hi1)
    zero = jnp.zeros_like(o_ref)
    p0 = jnp.where(m0, p_ref[0], zero)
    p1 = jnp.where(m1, p_ref[1], zero)
    o_ref[...] = (p0 + p1) * inv_ref[0]


_COMBINE_BLK = 1000


def _combine(partials, rng, inv_arr):
    return pl.pallas_call(
        _combine_body,
        grid=(NUM_NODES // _COMBINE_BLK,),
        in_specs=[
            pl.BlockSpec(memory_space=pltpu.SMEM),
            pl.BlockSpec(memory_space=pltpu.SMEM),
            pl.BlockSpec((NC, _COMBINE_BLK, D), lambda i: (0, i, 0)),
        ],
        out_specs=pl.BlockSpec((_COMBINE_BLK, D), lambda i: (i, 0)),
        out_shape=jax.ShapeDtypeStruct((NUM_NODES, D), jnp.float32),
    )(inv_arr, rng, partials)


def kernel(sh_vectors, cutoffs, receivers, inv_avg_num_neighbors):
    recv32 = receivers.astype(jnp.int32)
    cut_flat = cutoffs.reshape(NUM_EDGES)
    partials, rng = _sc_scatter(sh_vectors, cut_flat, recv32)
    inv_arr = jnp.reshape(inv_avg_num_neighbors, (1,)).astype(jnp.float32)
    return _combine(partials.reshape(NC, NUM_NODES, D), rng, inv_arr)
